# baseline (speedup 1.0000x reference)
import jax
import jax.numpy as jnp
from jax.experimental import pallas as pl
from jax.experimental.pallas import tpu as pltpu

_B = 128
_D = 64
_BLOCK = 16384


def _flash_kernel(x_ref, k1_ref, k2_ref, v1_ref, v2_ref, o_ref,
                  m_ref, l_ref, acc_ref):
    i = pl.program_id(0)
    n = pl.num_programs(0)

    @pl.when(i == 0)
    def _init():
        m_ref[...] = jnp.full_like(m_ref, -jnp.inf)
        l_ref[...] = jnp.zeros_like(l_ref)
        acc_ref[...] = jnp.zeros_like(acc_ref)

    x = x_ref[...]
    s1 = jax.lax.dot_general(
        x, k1_ref[...], (((1,), (0,)), ((), ())),
        preferred_element_type=jnp.float32)
    s2 = jax.lax.dot_general(
        x, k2_ref[...], (((1,), (0,)), ((), ())),
        preferred_element_type=jnp.float32)

    m_prev = m_ref[...]
    m_cur = jnp.maximum(jnp.max(s1, axis=1, keepdims=True),
                        jnp.max(s2, axis=1, keepdims=True))
    m_new = jnp.maximum(m_prev, m_cur)

    alpha = jnp.exp(m_prev - m_new)
    p1 = jnp.exp(s1 - m_new[:, 0:1])
    p2 = jnp.exp(s2 - m_new[:, 0:1])

    l_cur = (jnp.sum(p1, axis=1, keepdims=True)
             + jnp.sum(p2, axis=1, keepdims=True))
    l_ref[...] = l_ref[...] * alpha + l_cur
    m_ref[...] = m_new

    pv = (jax.lax.dot_general(
              p1, v1_ref[...], (((1,), (1,)), ((), ())),
              preferred_element_type=jnp.float32)
          + jax.lax.dot_general(
              p2, v2_ref[...], (((1,), (1,)), ((), ())),
              preferred_element_type=jnp.float32))
    acc_ref[...] = acc_ref[...] * alpha[:, 0:1] + pv

    @pl.when(i == n - 1)
    def _finish():
        o_ref[...] = acc_ref[...] / l_ref[...][:, 0:1]


def kernel(x, memory_keys, memory_values):
    kT = memory_keys.T
    vT = memory_values.T
    m_total = memory_keys.shape[0]
    n = (m_total // _BLOCK) // 2
    return pl.pallas_call(
        _flash_kernel,
        grid=(n,),
        in_specs=[
            pl.BlockSpec((_B, _D), lambda i: (0, 0)),
            pl.BlockSpec((_D, _BLOCK), lambda i: (0, i)),
            pl.BlockSpec((_D, _BLOCK), lambda i, _n=n: (0, i + _n)),
            pl.BlockSpec((_D, _BLOCK), lambda i: (0, i)),
            pl.BlockSpec((_D, _BLOCK), lambda i, _n=n: (0, i + _n)),
        ],
        out_specs=pl.BlockSpec((_B, _D), lambda i: (0, 0)),
        out_shape=jax.ShapeDtypeStruct((_B, _D), jnp.float32),
        scratch_shapes=[
            pltpu.VMEM((_B, 128), jnp.float32),
            pltpu.VMEM((_B, 128), jnp.float32),
            pltpu.VMEM((_B, _D), jnp.float32),
        ],
        compiler_params=pltpu.CompilerParams(
            dimension_semantics=("parallel",),
        ),
    )(x, kT, kT, vT, vT)
